# baseline (speedup 1.0000x reference)
import functools

import jax
import jax.numpy as jnp
from jax import lax
from jax.experimental import pallas as pl
from jax.experimental.pallas import tpu as pltpu
from jax.experimental.pallas import tpu_sc as plsc

N = 10000
E = 160000
D = 256
DEPTH = 3
H = 128

NC = 2
NS = 16
K = 128
CHUNKS = 80
PIPE = 2
HALVES = 2
EP = NS * CHUNKS * K
IDX_ROWS = EP // K
ACC_ROWS = 10240
ZERO_PER_TILE = ACC_ROWS // NS
OUT_PER_TILE = 1000


def _seg_sum_sc(h_stacked, src0, src1, dst2, zeros):
  mesh = plsc.VectorSubcoreMesh(core_axis_name="c", subcore_axis_name="s",
                                num_cores=NC, num_subcores=NS)

  @functools.partial(
      pl.kernel,
      mesh=mesh,
      out_type=jax.ShapeDtypeStruct((2 * N, H), jnp.float32),
      scratch_types=[
          pltpu.VMEM((CHUNKS // HALVES, K), jnp.int32),
          pltpu.VMEM((CHUNKS // HALVES, K), jnp.int32),
      ] + [pltpu.VMEM((K, H), jnp.float32)] * PIPE
      + [pltpu.VMEM_SHARED((ACC_ROWS, H), jnp.float32)]
      + [pltpu.SemaphoreType.DMA] * (2 * PIPE),
  )
  def k(h_hbm, src0_hbm, src1_hbm, dst_hbm, zeros_hbm, out_hbm,
        src_v, dst_v, *rest):
    rows = rest[:PIPE]
    acc = rest[PIPE]
    gsem = rest[PIPE + 1:2 * PIPE + 1]
    ssem = rest[2 * PIPE + 1:]
    cid = lax.axis_index("c")
    sid = lax.axis_index("s")
    hr = CHUNKS // HALVES

    pltpu.sync_copy(zeros_hbm.at[pl.ds(sid * ZERO_PER_TILE, ZERO_PER_TILE)],
                    acc.at[pl.ds(sid * ZERO_PER_TILE, ZERO_PER_TILE)])
    plsc.subcore_barrier()

    def gather(j, b):
      pltpu.async_copy(h_hbm.at[src_v.at[j]], rows[b], gsem[b])

    def wait_gather(j, b):
      pltpu.make_async_copy(h_hbm.at[src_v.at[j]], rows[b], gsem[b]).wait()

    def scatter(j, b):
      pltpu.async_copy(rows[b], acc.at[dst_v.at[j]], ssem[b], add=True)

    def wait_scatter(j, b):
      pltpu.make_async_copy(rows[b], acc.at[dst_v.at[j]], ssem[b]).wait()

    for half in range(HALVES):
      base = sid * CHUNKS + half * hr
      @pl.when(cid == 0)
      def _():
        pltpu.sync_copy(src0_hbm.at[pl.ds(base, hr)], src_v)

      @pl.when(cid != 0)
      def _():
        pltpu.sync_copy(src1_hbm.at[pl.ds(base, hr)], src_v)

      pltpu.sync_copy(dst_hbm.at[pl.ds(base, hr)], dst_v)

      gather(0, 0)

      def step(j, bcur, bnext):
        @pl.when(j >= 1)
        def _():
          wait_scatter(j - 1, bnext)

        @pl.when(j + 1 < hr)
        def _():
          gather(j + 1, bnext)

        wait_gather(j, bcur)
        scatter(j, bcur)

      def body(t, carry):
        step(2 * t, 0, 1)
        step(2 * t + 1, 1, 0)
        return carry

      lax.fori_loop(0, hr // 2, body, 0)
      wait_scatter(hr - 1, (hr - 1) % 2)
    plsc.subcore_barrier()

    @pl.when(sid < 10)
    def _():
      pltpu.sync_copy(
          acc.at[pl.ds(sid * OUT_PER_TILE, OUT_PER_TILE)],
          out_hbm.at[pl.ds(cid * N + sid * OUT_PER_TILE, OUT_PER_TILE)])

  return k(h_stacked, src0, src1, dst2, zeros)


RB = 2000
NB = N // RB
def _mm_h_kernel(x_ref, w_ref, o_ref):
  o_ref[...] = jnp.dot(x_ref[...], w_ref[...],
                       preferred_element_type=jnp.float32)


def _mm_h(x, w0):
  return pl.pallas_call(
      _mm_h_kernel,
      grid=(NC, NB),
      in_specs=[
          pl.BlockSpec((RB, D), lambda c, i: (i, 0)),
          pl.BlockSpec((D, H), lambda c, i: (0, c)),
      ],
      out_specs=pl.BlockSpec((RB, H), lambda c, i: (c * NB + i, 0)),
      out_shape=jax.ShapeDtypeStruct((2 * N, H), jnp.float32),
  )(x, w0)


def _combine_kernel(x_ref, ma_ref, mb_ref, wf_ref, wa_ref, wb_ref, o_ref):
  y = jnp.dot(x_ref[...], wf_ref[...], preferred_element_type=jnp.float32)
  y += jnp.dot(ma_ref[...], wa_ref[...], preferred_element_type=jnp.float32)
  y += jnp.dot(mb_ref[...], wb_ref[...], preferred_element_type=jnp.float32)
  mu = jnp.mean(y, axis=1, keepdims=True)
  d = y - mu
  var = jnp.mean(d * d, axis=1, keepdims=True)
  o_ref[...] = y * lax.rsqrt(var)


def _combine(x, msg, wf, w2b0, w2b1):
  return pl.pallas_call(
      _combine_kernel,
      grid=(NB,),
      in_specs=[
          pl.BlockSpec((RB, D), lambda i: (i, 0)),
          pl.BlockSpec((RB, H), lambda i: (i, 0)),
          pl.BlockSpec((RB, H), lambda i: (i + NB, 0)),
          pl.BlockSpec((D, D), lambda i: (0, 0)),
          pl.BlockSpec((H, D), lambda i: (0, 0)),
          pl.BlockSpec((H, D), lambda i: (0, 0)),
      ],
      out_specs=pl.BlockSpec((RB, D), lambda i: (i, 0)),
      out_shape=jax.ShapeDtypeStruct((N, D), jnp.float32),
  )(x, msg, msg, wf, w2b0, w2b1)


def _prep_kernel(w1_ref, w2_ref, o_ref):
  o_ref[0] = jnp.dot(w1_ref[0], w2_ref[0],
                     preferred_element_type=jnp.float32)


def _prep(w1, w2a):
  return pl.pallas_call(
      _prep_kernel,
      grid=(DEPTH,),
      in_specs=[
          pl.BlockSpec((1, D, D), lambda i: (i, 0, 0)),
          pl.BlockSpec((1, D, D), lambda i: (i, 0, 0)),
      ],
      out_specs=pl.BlockSpec((1, D, D), lambda i: (i, 0, 0)),
      out_shape=jax.ShapeDtypeStruct((DEPTH, D, D), jnp.float32),
  )(w1, w2a)


def kernel(x, edge_index, W0, W1, W2):
  src = edge_index[0].astype(jnp.int32)
  dst = edge_index[1].astype(jnp.int32)
  order = jnp.argsort(src)
  src = src[order]
  dst = dst[order]
  pad = EP - E
  pad_ar = jnp.arange(pad, dtype=jnp.int32)
  src_p = jnp.concatenate([src, (pad_ar * 61) % N])
  dst_p = jnp.concatenate([dst, N + pad_ar % (ACC_ROWS - N)])
  src0 = src_p.reshape(IDX_ROWS, K)
  src1 = src0 + N
  dst2 = dst_p.reshape(IDX_ROWS, K)
  zeros = jnp.zeros((ACC_ROWS, H), jnp.float32)

  wf = _prep(W1, W2[:, :D, :])
  w2b0 = W2[:, D:D + H, :]
  w2b1 = W2[:, D + H:, :]

  for i in range(DEPTH):
    h_stacked = _mm_h(x, W0[i])
    msg = _seg_sum_sc(h_stacked, src0, src1, dst2, zeros)
    x = _combine(x, msg, wf[i], w2b0[i], w2b1[i])
  return x


# speedup vs baseline: 1.9383x; 1.9383x over previous
import functools

import jax
import jax.numpy as jnp
from jax import lax
from jax.experimental import pallas as pl
from jax.experimental.pallas import tpu as pltpu
from jax.experimental.pallas import tpu_sc as plsc

N = 10000
E = 160000
D = 256
DEPTH = 3
H = 128

NC = 2
NS = 16
K = 128
CHUNKS = 80
PIPE = 2
HALVES = 2
EP = NS * CHUNKS * K
IDX_ROWS = EP // K
ACC_ROWS = 10240
ZERO_PER_TILE = ACC_ROWS // NS
OUT_PER_TILE = 1000


def _seg_sum_sc(h_stacked, src0, src1, dst2, zeros):
  mesh = plsc.VectorSubcoreMesh(core_axis_name="c", subcore_axis_name="s",
                                num_cores=NC, num_subcores=NS)

  @functools.partial(
      pl.kernel,
      mesh=mesh,
      out_type=jax.ShapeDtypeStruct((2 * N, H), jnp.float32),
      scratch_types=[
          pltpu.VMEM((CHUNKS // HALVES, K), jnp.int32),
          pltpu.VMEM((CHUNKS // HALVES, K), jnp.int32),
      ] + [pltpu.VMEM((K, H), jnp.float32)] * PIPE
      + [pltpu.VMEM_SHARED((ACC_ROWS, H), jnp.float32)]
      + [pltpu.SemaphoreType.DMA] * (2 * PIPE),
  )
  def k(h_hbm, src0_hbm, src1_hbm, dst_hbm, zeros_hbm, out_hbm,
        src_v, dst_v, *rest):
    rows = rest[:PIPE]
    acc = rest[PIPE]
    gsem = rest[PIPE + 1:2 * PIPE + 1]
    ssem = rest[2 * PIPE + 1:]
    cid = lax.axis_index("c")
    sid = lax.axis_index("s")
    hr = CHUNKS // HALVES

    pltpu.sync_copy(zeros_hbm.at[pl.ds(sid * ZERO_PER_TILE, ZERO_PER_TILE)],
                    acc.at[pl.ds(sid * ZERO_PER_TILE, ZERO_PER_TILE)])
    plsc.subcore_barrier()

    def gather(j, b):
      pltpu.async_copy(h_hbm.at[src_v.at[j]], rows[b], gsem[b])

    def wait_gather(j, b):
      pltpu.make_async_copy(h_hbm.at[src_v.at[j]], rows[b], gsem[b]).wait()

    def scatter(j, b):
      pltpu.async_copy(rows[b], acc.at[dst_v.at[j]], ssem[b], add=True)

    def wait_scatter(j, b):
      pltpu.make_async_copy(rows[b], acc.at[dst_v.at[j]], ssem[b]).wait()

    for half in range(HALVES):
      base = sid * CHUNKS + half * hr
      @pl.when(cid == 0)
      def _():
        pltpu.sync_copy(src0_hbm.at[pl.ds(base, hr)], src_v)

      @pl.when(cid != 0)
      def _():
        pltpu.sync_copy(src1_hbm.at[pl.ds(base, hr)], src_v)

      pltpu.sync_copy(dst_hbm.at[pl.ds(base, hr)], dst_v)

      gather(0, 0)

      def step(j, bcur, bnext):
        @pl.when(j >= 1)
        def _():
          wait_scatter(j - 1, bnext)

        @pl.when(j + 1 < hr)
        def _():
          gather(j + 1, bnext)

        wait_gather(j, bcur)
        scatter(j, bcur)

      def body(t, carry):
        step(2 * t, 0, 1)
        step(2 * t + 1, 1, 0)
        return carry

      lax.fori_loop(0, hr // 2, body, 0)
      wait_scatter(hr - 1, (hr - 1) % 2)
    plsc.subcore_barrier()

    @pl.when(sid < 10)
    def _():
      pltpu.sync_copy(
          acc.at[pl.ds(sid * OUT_PER_TILE, OUT_PER_TILE)],
          out_hbm.at[pl.ds(cid * N + sid * OUT_PER_TILE, OUT_PER_TILE)])

  return k(h_stacked, src0, src1, dst2, zeros)


RB = 2000
NB = N // RB
def _mm_h_kernel(x_ref, w_ref, o_ref):
  o_ref[...] = jnp.dot(x_ref[...], w_ref[...],
                       preferred_element_type=jnp.float32)


def _mm_h(x, w0):
  return pl.pallas_call(
      _mm_h_kernel,
      grid=(NC, NB),
      in_specs=[
          pl.BlockSpec((RB, D), lambda c, i: (i, 0)),
          pl.BlockSpec((D, H), lambda c, i: (0, c)),
      ],
      out_specs=pl.BlockSpec((RB, H), lambda c, i: (c * NB + i, 0)),
      out_shape=jax.ShapeDtypeStruct((2 * N, H), jnp.float32),
  )(x, w0)


def _combine_kernel(x_ref, ma_ref, mb_ref, wf_ref, wa_ref, wb_ref, o_ref):
  y = jnp.dot(x_ref[...], wf_ref[...], preferred_element_type=jnp.float32)
  y += jnp.dot(ma_ref[...], wa_ref[...], preferred_element_type=jnp.float32)
  y += jnp.dot(mb_ref[...], wb_ref[...], preferred_element_type=jnp.float32)
  mu = jnp.mean(y, axis=1, keepdims=True)
  d = y - mu
  var = jnp.mean(d * d, axis=1, keepdims=True)
  o_ref[...] = y * lax.rsqrt(var)


def _combine(x, msg, wf, w2b0, w2b1):
  return pl.pallas_call(
      _combine_kernel,
      grid=(NB,),
      in_specs=[
          pl.BlockSpec((RB, D), lambda i: (i, 0)),
          pl.BlockSpec((RB, H), lambda i: (i, 0)),
          pl.BlockSpec((RB, H), lambda i: (i + NB, 0)),
          pl.BlockSpec((D, D), lambda i: (0, 0)),
          pl.BlockSpec((H, D), lambda i: (0, 0)),
          pl.BlockSpec((H, D), lambda i: (0, 0)),
      ],
      out_specs=pl.BlockSpec((RB, D), lambda i: (i, 0)),
      out_shape=jax.ShapeDtypeStruct((N, D), jnp.float32),
  )(x, msg, msg, wf, w2b0, w2b1)


def _prep_kernel(w1_ref, w2_ref, o_ref):
  o_ref[0] = jnp.dot(w1_ref[0], w2_ref[0],
                     preferred_element_type=jnp.float32)


def _prep(w1, w2a):
  return pl.pallas_call(
      _prep_kernel,
      grid=(DEPTH,),
      in_specs=[
          pl.BlockSpec((1, D, D), lambda i: (i, 0, 0)),
          pl.BlockSpec((1, D, D), lambda i: (i, 0, 0)),
      ],
      out_specs=pl.BlockSpec((1, D, D), lambda i: (i, 0, 0)),
      out_shape=jax.ShapeDtypeStruct((DEPTH, D, D), jnp.float32),
  )(w1, w2a)


def kernel(x, edge_index, W0, W1, W2):
  src = edge_index[0].astype(jnp.int32)
  dst = edge_index[1].astype(jnp.int32)
  pad = EP - E
  pad_ar = jnp.arange(pad, dtype=jnp.int32)
  src_p = jnp.concatenate([src, (pad_ar * 61) % N])
  dst_p = jnp.concatenate([dst, N + pad_ar % (ACC_ROWS - N)])
  src0 = src_p.reshape(IDX_ROWS, K)
  src1 = src0 + N
  dst2 = dst_p.reshape(IDX_ROWS, K)
  zeros = jnp.zeros((ACC_ROWS, H), jnp.float32)

  wf = _prep(W1, W2[:, :D, :])
  w2b0 = W2[:, D:D + H, :]
  w2b1 = W2[:, D + H:, :]

  for i in range(DEPTH):
    h_stacked = _mm_h(x, W0[i])
    msg = _seg_sum_sc(h_stacked, src0, src1, dst2, zeros)
    x = _combine(x, msg, wf[i], w2b0[i], w2b1[i])
  return x
